# baseline (speedup 1.0000x reference)
import functools

import jax
import jax.numpy as jnp
from jax import lax
from jax.experimental import pallas as pl
from jax.experimental.pallas import tpu as pltpu
from jax.experimental.pallas import tpu_sc as plsc

N = 36864
D = 64
K = 1024


TB = 1024
G = N // TB


KC = 128
NKC = K // KC


def _argmin_body(x_ref, e_ref, out_ref):
    x = x_ref[...]
    s1 = jnp.sum(x * x, axis=1, keepdims=True)
    lane = lax.broadcasted_iota(jnp.int32, (TB, KC), 1)
    best_v = None
    for kc in range(NKC):
        e = e_ref[pl.ds(kc * KC, KC), :]
        xe = lax.dot_general(
            x, e, (((1,), (1,)), ((), ())),
            preferred_element_type=jnp.float32,
        )
        s2 = jnp.sum(e * e, axis=1)[None, :]
        dist = -(s1 - 2.0 * xe + s2)
        idc = lane + (kc * KC)
        if best_v is None:
            best_v, best_i = dist, idc
        else:
            take = dist > best_v
            best_v = jnp.where(take, dist, best_v)
            best_i = jnp.where(take, idc, best_i)
    m = jnp.max(best_v, axis=1, keepdims=True)
    cand = jnp.where(best_v == m, best_i, jnp.int32(1 << 30))
    out_ref[0, 0, :] = jnp.min(cand, axis=1)


def _argmin_indices(flat, embed, g0, g1):
    gs = g1 - g0
    return pl.pallas_call(
        _argmin_body,
        grid=(gs,),
        in_specs=[
            pl.BlockSpec((TB, D), lambda i: (i + g0, 0)),
            pl.BlockSpec((K, D), lambda i: (0, 0)),
        ],
        out_specs=pl.BlockSpec((1, 1, TB), lambda i: (i, 0, 0)),
        out_shape=jax.ShapeDtypeStruct((gs, 1, TB), jnp.int32),
    )(flat, embed)


NC = 2
NS = 16
NW = NC * NS
CHUNK = 128
ROWS_PER_W = N // NW
CHUNKS_PER_W = ROWS_PER_W // CHUNK

@functools.cache
def _sc_gather_fn(n_tok, chunk, cpw):
    rows_per_w = n_tok // NW

    def body(table_hbm, idx_hbm, out_hbm, idx_v, rows_v, sem):
        wid = lax.axis_index("s") * NC + lax.axis_index("c")
        pltpu.sync_copy(idx_hbm.at[wid], idx_v)
        copies = []
        for j in range(cpw):
            copies.append(pltpu.async_copy(
                table_hbm.at[idx_v.at[j]],
                rows_v.at[pl.ds(j * chunk, chunk)],
                sem,
            ))
        for c in copies:
            c.wait()
        pltpu.sync_copy(rows_v, out_hbm.at[pl.ds(wid * rows_per_w, rows_per_w)])

    mesh = plsc.VectorSubcoreMesh(
        core_axis_name="c", subcore_axis_name="s",
        num_cores=NC, num_subcores=NS)
    return pl.kernel(
        body,
        out_type=jax.ShapeDtypeStruct((n_tok, D), jnp.float32),
        mesh=mesh,
        scratch_types=[
            pltpu.VMEM((cpw, chunk), jnp.int32),
            pltpu.VMEM((rows_per_w, D), jnp.float32),
            pltpu.SemaphoreType.DMA,
        ],
        compiler_params=pltpu.CompilerParams(use_tc_tiling_on_sc=False),
    )


NSLICE = 2
GS = G // NSLICE
NSL = N // NSLICE
SL_CHUNK = 96
SL_CPW = NSL // NW // SL_CHUNK


def kernel(x, embed):
    shape = x.shape
    flat = x.reshape(-1, shape[-1])
    outs = []
    idxs = [
        _argmin_indices(flat, embed, s * GS, (s + 1) * GS)
        .reshape(NW, SL_CPW, SL_CHUNK)
        for s in range(NSLICE)
    ]
    gather = _sc_gather_fn(NSL, SL_CHUNK, SL_CPW)
    outs = [gather(embed, idx) for idx in idxs]
    out = jnp.concatenate(outs, axis=0)
    return out.reshape(shape)


# speedup vs baseline: 1.0422x; 1.0422x over previous
import functools

import jax
import jax.numpy as jnp
from jax import lax
from jax.experimental import pallas as pl
from jax.experimental.pallas import tpu as pltpu
from jax.experimental.pallas import tpu_sc as plsc

N = 36864
D = 64
K = 1024


TB = 1024
G = N // TB


KC = 128
NKC = K // KC


def _argmin_body(x_ref, e_ref, out_ref):
    x = x_ref[...]
    s1 = jnp.sum(x * x, axis=1, keepdims=True)
    lane = lax.broadcasted_iota(jnp.int32, (TB, KC), 1)
    best_v = None
    for kc in range(NKC):
        e = e_ref[pl.ds(kc * KC, KC), :]
        xe = lax.dot_general(
            x, e, (((1,), (1,)), ((), ())),
            preferred_element_type=jnp.float32,
        )
        s2 = jnp.sum(e * e, axis=1)[None, :]
        dist = -(s1 - 2.0 * xe + s2)
        idc = lane + (kc * KC)
        if best_v is None:
            best_v, best_i = dist, idc
        else:
            take = dist > best_v
            best_v = jnp.where(take, dist, best_v)
            best_i = jnp.where(take, idc, best_i)
    m = jnp.max(best_v, axis=1, keepdims=True)
    cand = jnp.where(best_v == m, best_i, jnp.int32(1 << 30))
    out_ref[0, 0, :] = jnp.min(cand, axis=1)


def _argmin_indices(flat, embed, g0, g1):
    gs = g1 - g0
    return pl.pallas_call(
        _argmin_body,
        grid=(gs,),
        in_specs=[
            pl.BlockSpec((TB, D), lambda i: (i + g0, 0)),
            pl.BlockSpec((K, D), lambda i: (0, 0)),
        ],
        out_specs=pl.BlockSpec((1, 1, TB), lambda i: (i, 0, 0)),
        out_shape=jax.ShapeDtypeStruct((gs, 1, TB), jnp.int32),
    )(flat, embed)


NC = 2
NS = 16
NW = NC * NS
CHUNK = 128
ROWS_PER_W = N // NW
CHUNKS_PER_W = ROWS_PER_W // CHUNK

@functools.cache
def _sc_gather_fn(n_tok, chunk, cpw):
    rows_per_w = n_tok // NW

    def body(table_hbm, idx_hbm, out_hbm, idx_v, rows_v, sem):
        wid = lax.axis_index("s") * NC + lax.axis_index("c")
        pltpu.sync_copy(idx_hbm.at[wid], idx_v)
        copies = []
        for j in range(cpw):
            copies.append(pltpu.async_copy(
                table_hbm.at[idx_v.at[j]],
                rows_v.at[pl.ds(j * chunk, chunk)],
                sem,
            ))
        for c in copies:
            c.wait()
        pltpu.sync_copy(rows_v, out_hbm.at[pl.ds(wid * rows_per_w, rows_per_w)])

    mesh = plsc.VectorSubcoreMesh(
        core_axis_name="c", subcore_axis_name="s",
        num_cores=NC, num_subcores=NS)
    return pl.kernel(
        body,
        out_type=jax.ShapeDtypeStruct((n_tok, 2 * D), jnp.float32),
        mesh=mesh,
        scratch_types=[
            pltpu.VMEM((cpw, chunk), jnp.int32),
            pltpu.VMEM((rows_per_w, 2 * D), jnp.float32),
            pltpu.SemaphoreType.DMA,
        ],
        compiler_params=pltpu.CompilerParams(use_tc_tiling_on_sc=True),
    )


NSLICE = 3
GS = G // NSLICE
NSL = N // NSLICE
SL_CHUNK = 128
SL_CPW = NSL // NW // SL_CHUNK


def kernel(x, embed):
    shape = x.shape
    flat = x.reshape(-1, shape[-1])
    table = jnp.concatenate([embed, embed], axis=1)
    gather = _sc_gather_fn(NSL, SL_CHUNK, SL_CPW)
    outs = []
    for s in range(NSLICE):
        idx = (_argmin_indices(flat, embed, s * GS, (s + 1) * GS)
               .reshape(NW, SL_CPW, SL_CHUNK))
        outs.append(gather(table, idx))
    out = jnp.concatenate([o[:, :D] for o in outs], axis=0)
    return out.reshape(shape)


# speedup vs baseline: 1.3001x; 1.2475x over previous
import functools

import jax
import jax.numpy as jnp
from jax import lax
from jax.experimental import pallas as pl
from jax.experimental.pallas import tpu as pltpu
from jax.experimental.pallas import tpu_sc as plsc

N = 36864
D = 64
K = 1024


T1 = 576
BD0 = 4
TB = BD0 * T1
G = N // TB


def _argmin_body(xt_ref, et_ref, out_ref):
    et = et_ref[...]
    s2 = jnp.sum(et * et, axis=0)[:, None]
    ids = lax.broadcasted_iota(jnp.int32, (K, T1), 0)
    for s in range(BD0):
        xts = xt_ref[s]
        s1 = jnp.sum(xts * xts, axis=0)[None, :]
        xe = lax.dot_general(
            et, xts, (((0,), (0,)), ((), ())),
            preferred_element_type=jnp.float32,
        )
        dist = -(s1 - 2.0 * xe + s2)
        m = jnp.max(dist, axis=0)[None, :]
        cand = jnp.where(dist == m, ids, jnp.int32(1 << 30))
        out_ref[0, 0, pl.ds(s * T1, T1)] = jnp.min(cand, axis=0)


def _argmin_indices(xt, et, g0, g1):
    gs = g1 - g0
    return pl.pallas_call(
        _argmin_body,
        grid=(gs,),
        in_specs=[
            pl.BlockSpec((BD0, D, T1), lambda i: (i + g0, 0, 0)),
            pl.BlockSpec((D, K), lambda i: (0, 0)),
        ],
        out_specs=pl.BlockSpec((1, 1, TB), lambda i: (i, 0, 0)),
        out_shape=jax.ShapeDtypeStruct((gs, 1, TB), jnp.int32),
    )(xt, et)


NC = 2
NS = 16
NW = NC * NS
CHUNK = 128
ROWS_PER_W = N // NW
CHUNKS_PER_W = ROWS_PER_W // CHUNK

@functools.cache
def _sc_gather_fn(n_tok, chunk, cpw):
    rows_per_w = n_tok // NW

    def body(table_hbm, idx_hbm, out_hbm, idx_v, rows_v, sem):
        wid = lax.axis_index("s") * NC + lax.axis_index("c")
        pltpu.sync_copy(idx_hbm.at[wid], idx_v)
        copies = []
        for j in range(cpw):
            copies.append(pltpu.async_copy(
                table_hbm.at[idx_v.at[j]],
                rows_v.at[pl.ds(j * chunk, chunk)],
                sem,
            ))
        for c in copies:
            c.wait()
        pltpu.sync_copy(rows_v, out_hbm.at[pl.ds(wid * rows_per_w, rows_per_w)])

    mesh = plsc.VectorSubcoreMesh(
        core_axis_name="c", subcore_axis_name="s",
        num_cores=NC, num_subcores=NS)
    return pl.kernel(
        body,
        out_type=jax.ShapeDtypeStruct((n_tok, 2 * D), jnp.float32),
        mesh=mesh,
        scratch_types=[
            pltpu.VMEM((cpw, chunk), jnp.int32),
            pltpu.VMEM((rows_per_w, 2 * D), jnp.float32),
            pltpu.SemaphoreType.DMA,
        ],
        compiler_params=pltpu.CompilerParams(use_tc_tiling_on_sc=True),
    )


NSLICE = 4
GS = G // NSLICE
NSL = N // NSLICE
SL_CHUNK = 96
SL_CPW = NSL // NW // SL_CHUNK


def kernel(x, embed):
    shape = x.shape
    xt = x.transpose(0, 2, 1)
    et = embed.T
    table = jnp.concatenate([embed, embed], axis=1)
    gather = _sc_gather_fn(NSL, SL_CHUNK, SL_CPW)
    outs = []
    for s in range(NSLICE):
        idx = (_argmin_indices(xt, et, s * GS, (s + 1) * GS)
               .reshape(NW, SL_CPW, SL_CHUNK))
        outs.append(gather(table, idx))
    out = jnp.concatenate([o[:, :D] for o in outs], axis=0)
    return out.reshape(shape)


# speedup vs baseline: 1.3793x; 1.0609x over previous
import functools

import jax
import jax.numpy as jnp
from jax import lax
from jax.experimental import pallas as pl
from jax.experimental.pallas import tpu as pltpu
from jax.experimental.pallas import tpu_sc as plsc

N = 36864
D = 64
K = 1024


T1 = 576
BD0 = 4
TB = BD0 * T1
G = N // TB


def _argmin_body(xt_ref, et_ref, out_ref):
    et = et_ref[...]
    s2 = jnp.sum(et * et, axis=0)[:, None]
    ids = lax.broadcasted_iota(jnp.int32, (K, T1), 0)
    for s in range(BD0):
        xts = xt_ref[s]
        s1 = jnp.sum(xts * xts, axis=0)[None, :]
        xe = lax.dot_general(
            et, xts, (((0,), (0,)), ((), ())),
            preferred_element_type=jnp.float32,
        )
        dist = -(s1 - 2.0 * xe + s2)
        m = jnp.max(dist, axis=0)[None, :]
        cand = jnp.where(dist == m, ids, jnp.int32(1 << 30))
        out_ref[0, 0, pl.ds(s * T1, T1)] = jnp.min(cand, axis=0)


def _argmin_indices(xt, et, g0, g1):
    gs = g1 - g0
    return pl.pallas_call(
        _argmin_body,
        grid=(gs,),
        in_specs=[
            pl.BlockSpec((BD0, D, T1), lambda i: (i + g0, 0, 0)),
            pl.BlockSpec((D, K), lambda i: (0, 0)),
        ],
        out_specs=pl.BlockSpec((1, 1, TB), lambda i: (i, 0, 0)),
        out_shape=jax.ShapeDtypeStruct((gs, 1, TB), jnp.int32),
    )(xt, et)


NC = 2
NS = 16
NW = NC * NS
CHUNK = 128
ROWS_PER_W = N // NW
CHUNKS_PER_W = ROWS_PER_W // CHUNK

@functools.cache
def _sc_gather_fn(n_tok, chunk, cpw, base):
    rows_per_w = n_tok // NW

    def body(table_hbm, idx_hbm, out_hbm, idx_v, rows_v, sem):
        wid = lax.axis_index("s") * NC + lax.axis_index("c")
        pltpu.sync_copy(idx_hbm.at[wid], idx_v)
        copies = []
        for j in range(cpw):
            copies.append(pltpu.async_copy(
                table_hbm.at[idx_v.at[j]],
                rows_v.at[pl.ds(j * chunk, chunk)],
                sem,
            ))
        for c in copies:
            c.wait()
        pltpu.sync_copy(
            rows_v, out_hbm.at[pl.ds(base + wid * rows_per_w, rows_per_w)])

    mesh = plsc.VectorSubcoreMesh(
        core_axis_name="c", subcore_axis_name="s",
        num_cores=NC, num_subcores=NS)
    return pl.kernel(
        body,
        out_type=(),
        mesh=mesh,
        scratch_types=[
            pltpu.VMEM((cpw, chunk), jnp.int32),
            pltpu.VMEM((rows_per_w, 2 * D), jnp.float32),
            pltpu.SemaphoreType.DMA,
        ],
        compiler_params=pltpu.CompilerParams(use_tc_tiling_on_sc=True),
    )


SLICE_GS = (5, 5, 5, 1)


def kernel(x, embed):
    shape = x.shape
    xt = x.transpose(0, 2, 1)
    et = embed.T
    table = jnp.concatenate([embed, embed], axis=1)
    out_ref = jax.new_ref(jnp.zeros((N, 2 * D), jnp.float32))
    g0 = 0
    for gs in SLICE_GS:
        n_tok = gs * TB
        rpw = n_tok // NW
        chunk = rpw // -(-rpw // 128)
        while rpw % chunk:
            chunk -= 1
        cpw = rpw // chunk
        idx = (_argmin_indices(xt, et, g0, g0 + gs)
               .reshape(NW, cpw, chunk))
        _sc_gather_fn(n_tok, chunk, cpw, g0 * TB)(table, idx, out_ref)
        g0 += gs
    out = out_ref[...]
    return out[:, :D].reshape(shape)


# speedup vs baseline: 1.4258x; 1.0337x over previous
import functools

import jax
import jax.numpy as jnp
from jax import lax
from jax.experimental import pallas as pl
from jax.experimental.pallas import tpu as pltpu
from jax.experimental.pallas import tpu_sc as plsc

N = 36864
D = 64
K = 1024


T1 = 576
BD0 = 4
TB = BD0 * T1
G = N // TB


def _argmin_body(xt_ref, et_ref, out_ref):
    et = et_ref[...]
    s2 = jnp.sum(et * et, axis=0)[:, None]
    ids = lax.broadcasted_iota(jnp.int32, (K, T1), 0)
    for s in range(BD0):
        xts = xt_ref[s]
        s1 = jnp.sum(xts * xts, axis=0)[None, :]
        xe = lax.dot_general(
            et, xts, (((0,), (0,)), ((), ())),
            preferred_element_type=jnp.float32,
        )
        dist = -(s1 - 2.0 * xe + s2)
        m = jnp.max(dist, axis=0)[None, :]
        cand = jnp.where(dist == m, ids, jnp.int32(1 << 30))
        out_ref[0, 0, pl.ds(s * T1, T1)] = jnp.min(cand, axis=0)


def _argmin_indices(xt, et, g0, g1):
    gs = g1 - g0
    return pl.pallas_call(
        _argmin_body,
        grid=(gs,),
        in_specs=[
            pl.BlockSpec((BD0, D, T1), lambda i: (i + g0, 0, 0)),
            pl.BlockSpec((D, K), lambda i: (0, 0)),
        ],
        out_specs=pl.BlockSpec((1, 1, TB), lambda i: (i, 0, 0)),
        out_shape=jax.ShapeDtypeStruct((gs, 1, TB), jnp.int32),
    )(xt, et)


NC = 2
NS = 16
NW = NC * NS
CHUNK = 128
ROWS_PER_W = N // NW
CHUNKS_PER_W = ROWS_PER_W // CHUNK

@functools.cache
def _sc_gather_fn(n_tok, chunk, cpw, base):
    rows_per_w = n_tok // NW

    def body(table_hbm, idx_hbm, out_hbm, idx_v, rows_v, sem):
        wid = lax.axis_index("s") * NC + lax.axis_index("c")
        pltpu.sync_copy(idx_hbm.at[wid], idx_v)
        copies = []
        for j in range(cpw):
            copies.append(pltpu.async_copy(
                table_hbm.at[idx_v.at[j]],
                rows_v.at[pl.ds(j * chunk, chunk)],
                sem,
            ))
        for c in copies:
            c.wait()
        pltpu.sync_copy(
            rows_v, out_hbm.at[pl.ds(base + wid * rows_per_w, rows_per_w)])

    mesh = plsc.VectorSubcoreMesh(
        core_axis_name="c", subcore_axis_name="s",
        num_cores=NC, num_subcores=NS)
    return pl.kernel(
        body,
        out_type=(),
        mesh=mesh,
        scratch_types=[
            pltpu.VMEM((cpw, chunk), jnp.int32),
            pltpu.VMEM((rows_per_w, 2 * D), jnp.float32),
            pltpu.SemaphoreType.DMA,
        ],
        compiler_params=pltpu.CompilerParams(use_tc_tiling_on_sc=True),
    )


SLICE_GS = (5, 4, 4, 3)


def kernel(x, embed):
    shape = x.shape
    xt = x.transpose(0, 2, 1)
    et = embed.T
    table = jnp.concatenate([embed, embed], axis=1)
    out_ref = jax.new_ref(jnp.zeros((N, 2 * D), jnp.float32))
    g0 = 0
    for gs in SLICE_GS:
        n_tok = gs * TB
        rpw = n_tok // NW
        chunk = rpw // -(-rpw // 128)
        while rpw % chunk:
            chunk -= 1
        cpw = rpw // chunk
        idx = (_argmin_indices(xt, et, g0, g0 + gs)
               .reshape(NW, cpw, chunk))
        _sc_gather_fn(n_tok, chunk, cpw, g0 * TB)(table, idx, out_ref)
        g0 += gs
    out = out_ref[...]
    return out[:, :D].reshape(shape)
